# baseline (speedup 1.0000x reference)
import functools

import jax
import jax.numpy as jnp
from jax import lax
from jax.experimental import pallas as pl
from jax.experimental.pallas import tpu as pltpu
from jax.experimental.pallas import tpu_sc as plsc

N = 10000
D_FEAT = 128
D_HID = 64
HEADS = 8
NUM_CLASSES = 16
DTAB = D_HID + 2 * HEADS
DROW = D_HID + HEADS

NC = 2
NS = 16
CH = 80

_BLK = 1000
_GRID = N // _BLK


def _proj_common(h, asrc_ref, adst_ref, taba_ref, tabb_ref, bvec_ref,
                 smax_sc, dmax_sc):
    i = pl.program_id(0)
    s = jnp.dot(h, asrc_ref[...], preferred_element_type=jnp.float32)
    d = jnp.dot(h, adst_ref[...], preferred_element_type=jnp.float32)
    taba_ref[:, :D_HID] = h
    taba_ref[:, D_HID:] = jnp.concatenate([s, s], axis=1)
    tabb_ref[...] = jnp.concatenate([d, d], axis=1)
    bs = jnp.max(s, axis=0, keepdims=True)
    bd = jnp.max(d, axis=0, keepdims=True)

    @pl.when(i == 0)
    def _():
        smax_sc[...] = bs
        dmax_sc[...] = bd

    @pl.when(i > 0)
    def _():
        smax_sc[...] = jnp.maximum(smax_sc[...], bs)
        dmax_sc[...] = jnp.maximum(dmax_sc[...], bd)

    bb = smax_sc[...] + dmax_sc[...]
    bb = jnp.maximum(bb, 0.2 * bb)
    bvec_ref[...] = jnp.concatenate([bb, bb], axis=1)


def _tc_prep_body(x_ref, w_ref, asrc_ref, adst_ref,
                  taba_ref, tabb_ref, bvec_ref, smax_sc, dmax_sc):
    h = jnp.dot(x_ref[...], w_ref[...], preferred_element_type=jnp.float32)
    _proj_common(h, asrc_ref, adst_ref, taba_ref, tabb_ref, bvec_ref,
                 smax_sc, dmax_sc)


def _normalize_elu(parts, rexp_ref):
    num = parts[0] + parts[1]
    den = num[:, :HEADS]
    dexp = jnp.dot(den, rexp_ref[...], preferred_element_type=jnp.float32)
    z = num[:, HEADS:] / (dexp + 1e-16)
    return jnp.where(z > 0, z, jnp.exp(z) - 1.0)


def _tc_mid_body(parts_ref, w_ref, asrc_ref, adst_ref, rexp_ref,
                 taba_ref, tabb_ref, bvec_ref, smax_sc, dmax_sc):
    z = _normalize_elu(parts_ref[...], rexp_ref)
    h = jnp.dot(z, w_ref[...], preferred_element_type=jnp.float32)
    _proj_common(h, asrc_ref, adst_ref, taba_ref, tabb_ref, bvec_ref,
                 smax_sc, dmax_sc)


def _tc_final_body(parts_ref, wlin_ref, blin_ref, rexp_ref, out_ref):
    z = _normalize_elu(parts_ref[...], rexp_ref)
    out_ref[...] = (jnp.dot(z, wlin_ref[...], preferred_element_type=jnp.float32)
                    + blin_ref[...])


def _tc_prep(x, w, asrc, adst):
    return pl.pallas_call(
        _tc_prep_body,
        grid=(_GRID,),
        in_specs=[
            pl.BlockSpec((_BLK, D_FEAT), lambda i: (i, 0)),
            pl.BlockSpec((D_FEAT, D_HID), lambda i: (0, 0)),
            pl.BlockSpec((D_HID, HEADS), lambda i: (0, 0)),
            pl.BlockSpec((D_HID, HEADS), lambda i: (0, 0)),
        ],
        out_specs=[
            pl.BlockSpec((_BLK, DTAB), lambda i: (i, 0)),
            pl.BlockSpec((_BLK, 2 * HEADS), lambda i: (i, 0)),
            pl.BlockSpec((1, 2 * HEADS), lambda i: (0, 0)),
        ],
        out_shape=[
            jax.ShapeDtypeStruct((N, DTAB), jnp.float32),
            jax.ShapeDtypeStruct((N, 2 * HEADS), jnp.float32),
            jax.ShapeDtypeStruct((1, 2 * HEADS), jnp.float32),
        ],
        scratch_shapes=[
            pltpu.VMEM((1, HEADS), jnp.float32),
            pltpu.VMEM((1, HEADS), jnp.float32),
        ],
    )(x, w, asrc, adst)


def _tc_mid(parts, w, asrc, adst, rexp):
    return pl.pallas_call(
        _tc_mid_body,
        grid=(_GRID,),
        in_specs=[
            pl.BlockSpec((2, _BLK, DROW), lambda i: (0, i, 0)),
            pl.BlockSpec((D_HID, D_HID), lambda i: (0, 0)),
            pl.BlockSpec((D_HID, HEADS), lambda i: (0, 0)),
            pl.BlockSpec((D_HID, HEADS), lambda i: (0, 0)),
            pl.BlockSpec((HEADS, D_HID), lambda i: (0, 0)),
        ],
        out_specs=[
            pl.BlockSpec((_BLK, DTAB), lambda i: (i, 0)),
            pl.BlockSpec((_BLK, 2 * HEADS), lambda i: (i, 0)),
            pl.BlockSpec((1, 2 * HEADS), lambda i: (0, 0)),
        ],
        out_shape=[
            jax.ShapeDtypeStruct((N, DTAB), jnp.float32),
            jax.ShapeDtypeStruct((N, 2 * HEADS), jnp.float32),
            jax.ShapeDtypeStruct((1, 2 * HEADS), jnp.float32),
        ],
        scratch_shapes=[
            pltpu.VMEM((1, HEADS), jnp.float32),
            pltpu.VMEM((1, HEADS), jnp.float32),
        ],
    )(parts, w, asrc, adst, rexp)


def _tc_final(parts, wlin, blin2d, rexp):
    return pl.pallas_call(
        _tc_final_body,
        grid=(_GRID,),
        in_specs=[
            pl.BlockSpec((2, _BLK, DROW), lambda i: (0, i, 0)),
            pl.BlockSpec((D_HID, NUM_CLASSES), lambda i: (0, 0)),
            pl.BlockSpec((1, NUM_CLASSES), lambda i: (0, 0)),
            pl.BlockSpec((HEADS, D_HID), lambda i: (0, 0)),
        ],
        out_specs=pl.BlockSpec((_BLK, NUM_CLASSES), lambda i: (i, 0)),
        out_shape=jax.ShapeDtypeStruct((N, NUM_CLASSES), jnp.float32),
    )(parts, wlin, blin2d, rexp)


def _sc_edges_body(taba, tabb, src, dst, bvec, zeros, out,
                   acc, srcv0, srcv1, srcv2, srcv3, dstv0, dstv1, dstv2, dstv3,
                   bufa0, bufa1, bufa2, bufa3, bufb0, bufb1, bufb2, bufb3,
                   bufo0, bufo1, bufo2, bufo3, bvv,
                   sema0, sema1, sema2, sema3, semb0, semb1, semb2, semb3,
                   semo0, semo1, semo2, semo3, semi0, semi1, semi2, semi3):
    srcv = (srcv0, srcv1, srcv2, srcv3)
    dstv = (dstv0, dstv1, dstv2, dstv3)
    bufa = (bufa0, bufa1, bufa2, bufa3)
    bufb = (bufb0, bufb1, bufb2, bufb3)
    bufo = (bufo0, bufo1, bufo2, bufo3)
    sema = (sema0, sema1, sema2, sema3)
    semb = (semb0, semb1, semb2, semb3)
    semo = (semo0, semo1, semo2, semo3)
    semi = (semi0, semi1, semi2, semi3)
    E = src.shape[0]
    c = lax.axis_index("c")
    s = lax.axis_index("s")
    rows = (N // NS) & ~7
    tail = N - NS * rows

    per_tile = E // (NC * NS)
    nchunks = per_tile // CH

    pltpu.sync_copy(zeros.at[pl.ds(s * rows, rows)],
                    acc.at[pl.ds(s * rows, rows)])

    @pl.when(s == NS - 1)
    def _():
        pltpu.sync_copy(zeros.at[pl.ds(NS * rows, tail)],
                        acc.at[pl.ds(NS * rows, tail)])

    pltpu.sync_copy(bvec, bvv)
    plsc.subcore_barrier()

    bv = bvv[...]
    lane = lax.iota(jnp.int32, 16)
    hi1 = lane >> 3

    tile = c * NS + s
    base_t = tile * per_tile

    def start_idx(g, b):
        base = pl.multiple_of(base_t + g * CH, 8)
        pltpu.make_async_copy(src.at[pl.ds(base, CH)], srcv[b], semi[b]).start()
        pltpu.make_async_copy(dst.at[pl.ds(base, CH)], dstv[b], semi[b]).start()

    def wait_idx(b):
        pltpu.make_async_copy(src.at[pl.ds(base_t, CH)], srcv[b], semi[b]).wait()
        pltpu.make_async_copy(dst.at[pl.ds(base_t, CH)], dstv[b], semi[b]).wait()

    def start_gathers(b):
        pltpu.async_copy(taba.at[srcv[b]], bufa[b], sema[b])
        pltpu.async_copy(tabb.at[dstv[b]], bufb[b], semb[b])

    def wait_gathers(b):
        pltpu.make_async_copy(taba.at[srcv[b]], bufa[b], sema[b]).wait()
        pltpu.make_async_copy(tabb.at[dstv[b]], bufb[b], semb[b]).wait()

    def start_scatter(b):
        pltpu.make_async_copy(bufo[b], acc.at[dstv[b]], semo[b]).start(add=True)

    def wait_scatter(b):
        pltpu.make_async_copy(bufo[b], acc.at[dstv[b]], semo[b]).wait()

    def compute(b):
        ba, bb, bo = bufa[b], bufb[b], bufo[b]

        @plsc.parallel_loop(0, CH, unroll=4)
        def edge(e):
            sv = ba[e, pl.ds(D_HID, 16)]
            dv = bb[e, pl.ds(0, 16)]
            ev = sv + dv
            ev = jnp.maximum(ev, 0.2 * ev)
            exv = jnp.exp(ev - bv)
            prod0 = None
            for j in range(4):
                hh = ba[e, pl.ds(j * 16, 16)]
                mv = exv.at[2 * j + hi1].get(mode="promise_in_bounds")
                prod = hh * mv
                if j == 0:
                    prod0 = prod
                bo[e, pl.ds(8 + j * 16, 16)] = prod
            perm0 = prod0.at[lane & 7].get(mode="promise_in_bounds")
            bo[e, pl.ds(0, 16)] = jnp.where(lane < 8, exv, perm0)

    start_idx(0, 0)
    start_idx(1, 1)
    start_idx(2, 2)
    wait_idx(0)
    start_gathers(0)
    wait_idx(1)
    start_gathers(1)

    def slot_quad(g4, carry):
        for b4 in range(4):
            g = g4 * 4 + b4
            b = b4
            b2 = (b4 + 2) % 4
            b3 = (b4 + 3) % 4

            @pl.when(g < nchunks)
            def _():
                @pl.when(g >= 1)
                def _():
                    wait_scatter(b3)

                @pl.when(g + 3 < nchunks)
                def _():
                    start_idx(g + 3, b3)

                @pl.when(g + 2 < nchunks)
                def _():
                    wait_idx(b2)
                    start_gathers(b2)

                wait_gathers(b)
                compute(b)
                start_scatter(b)
        return carry

    lax.fori_loop(0, (nchunks + 3) // 4, slot_quad, 0)
    wait_scatter((nchunks - 1) % 4)
    plsc.subcore_barrier()
    pltpu.sync_copy(acc.at[pl.ds(s * rows, rows)],
                    out.at[c, pl.ds(s * rows, rows)])

    @pl.when(s == NS - 1)
    def _():
        pltpu.sync_copy(acc.at[pl.ds(NS * rows, tail)],
                        out.at[c, pl.ds(NS * rows, tail)])


def _sc_edges(taba, tabb, src, dst, bvec, zeros):
    mesh = plsc.VectorSubcoreMesh(core_axis_name="c", subcore_axis_name="s")
    kfn = functools.partial(
        pl.kernel,
        out_type=jax.ShapeDtypeStruct((NC, N, DROW), jnp.float32),
        mesh=mesh,
        compiler_params=pltpu.CompilerParams(use_tc_tiling_on_sc=False),
        scratch_types=(
            [pltpu.VMEM_SHARED((N, DROW), jnp.float32)]
            + [pltpu.VMEM((CH,), jnp.int32)] * 8
            + [pltpu.VMEM((CH, DTAB), jnp.float32)] * 4
            + [pltpu.VMEM((CH, 2 * HEADS), jnp.float32)] * 4
            + [pltpu.VMEM((CH, DROW), jnp.float32)] * 4
            + [pltpu.VMEM((2 * HEADS,), jnp.float32)]
            + [pltpu.SemaphoreType.DMA] * 16
        ),
    )(_sc_edges_body)
    return kfn(taba, tabb, src, dst, bvec, zeros)


def _mix_mat(a):
    eye = jnp.eye(HEADS, dtype=jnp.float32)
    return (eye[:, None, :] * a[:, :, None]).reshape(D_HID, HEADS)


def kernel(x, edge_index, W1, a1_src, a1_dst, Wsem1, bsem1, qsem1,
           W2, a2_src, a2_dst, Wsem2, bsem2, qsem2, Wlin, blin):
    del Wsem1, bsem1, qsem1, Wsem2, bsem2, qsem2
    src = edge_index[0]
    dst = edge_index[1]
    zeros = jnp.zeros((N, DROW), jnp.float32)
    rexp = jnp.kron(jnp.eye(HEADS, dtype=jnp.float32),
                    jnp.ones((1, D_HID // HEADS), jnp.float32))

    taba1, tabb1, bvec1 = _tc_prep(x, W1, _mix_mat(a1_src), _mix_mat(a1_dst))
    parts1 = _sc_edges(taba1, tabb1, src, dst, bvec1.reshape(-1), zeros)
    taba2, tabb2, bvec2 = _tc_mid(parts1, W2, _mix_mat(a2_src),
                                  _mix_mat(a2_dst), rexp)
    parts2 = _sc_edges(taba2, tabb2, src, dst, bvec2.reshape(-1), zeros)
    return _tc_final(parts2, Wlin, blin.reshape(1, -1), rexp)


# speedup vs baseline: 28.9489x; 28.9489x over previous
import functools

import jax
import jax.numpy as jnp
from jax import lax
from jax.experimental import pallas as pl
from jax.experimental.pallas import tpu as pltpu
from jax.experimental.pallas import tpu_sc as plsc

N = 10000
D_FEAT = 128
D_HID = 64
HEADS = 8
NUM_CLASSES = 16
DTAB = D_HID + 2 * HEADS
DROW = D_HID + HEADS

NC = 2
NS = 16
CH = 80

_BLK = 1000
_GRID = N // _BLK


def _proj_common(h, asrc_ref, adst_ref, taba_ref, tabb_ref, bvec_ref,
                 smax_sc, dmax_sc):
    i = pl.program_id(0)
    s = jnp.dot(h, asrc_ref[...], preferred_element_type=jnp.float32)
    d = jnp.dot(h, adst_ref[...], preferred_element_type=jnp.float32)
    taba_ref[:, :D_HID] = h
    taba_ref[:, D_HID:] = jnp.concatenate([s, s], axis=1)
    tabb_ref[...] = jnp.concatenate([d, d], axis=1)
    bs = jnp.max(s, axis=0, keepdims=True)
    bd = jnp.max(d, axis=0, keepdims=True)

    @pl.when(i == 0)
    def _():
        smax_sc[...] = bs
        dmax_sc[...] = bd

    @pl.when(i > 0)
    def _():
        smax_sc[...] = jnp.maximum(smax_sc[...], bs)
        dmax_sc[...] = jnp.maximum(dmax_sc[...], bd)

    bb = smax_sc[...] + dmax_sc[...]
    bb = jnp.maximum(bb, 0.2 * bb)
    bvec_ref[...] = jnp.concatenate([bb, bb], axis=1)


def _tc_prep_body(x_ref, w_ref, asrc_ref, adst_ref,
                  taba_ref, tabb_ref, bvec_ref, smax_sc, dmax_sc):
    h = jnp.dot(x_ref[...], w_ref[...], preferred_element_type=jnp.float32)
    _proj_common(h, asrc_ref, adst_ref, taba_ref, tabb_ref, bvec_ref,
                 smax_sc, dmax_sc)


def _normalize_elu(parts, rexp_ref):
    num = parts[0] + parts[1]
    den = num[:, :HEADS]
    dexp = jnp.dot(den, rexp_ref[...], preferred_element_type=jnp.float32)
    z = num[:, HEADS:] / (dexp + 1e-16)
    return jnp.where(z > 0, z, jnp.exp(z) - 1.0)


def _tc_mid_body(parts_ref, w_ref, asrc_ref, adst_ref, rexp_ref,
                 taba_ref, tabb_ref, bvec_ref, smax_sc, dmax_sc):
    z = _normalize_elu(parts_ref[...], rexp_ref)
    h = jnp.dot(z, w_ref[...], preferred_element_type=jnp.float32)
    _proj_common(h, asrc_ref, adst_ref, taba_ref, tabb_ref, bvec_ref,
                 smax_sc, dmax_sc)


def _tc_final_body(parts_ref, wlin_ref, blin_ref, rexp_ref, out_ref):
    z = _normalize_elu(parts_ref[...], rexp_ref)
    out_ref[...] = (jnp.dot(z, wlin_ref[...], preferred_element_type=jnp.float32)
                    + blin_ref[...])


def _tc_prep(x, w, asrc, adst):
    return pl.pallas_call(
        _tc_prep_body,
        grid=(_GRID,),
        in_specs=[
            pl.BlockSpec((_BLK, D_FEAT), lambda i: (i, 0)),
            pl.BlockSpec((D_FEAT, D_HID), lambda i: (0, 0)),
            pl.BlockSpec((D_HID, HEADS), lambda i: (0, 0)),
            pl.BlockSpec((D_HID, HEADS), lambda i: (0, 0)),
        ],
        out_specs=[
            pl.BlockSpec((_BLK, DTAB), lambda i: (i, 0)),
            pl.BlockSpec((_BLK, 2 * HEADS), lambda i: (i, 0)),
            pl.BlockSpec((1, 2 * HEADS), lambda i: (0, 0)),
        ],
        out_shape=[
            jax.ShapeDtypeStruct((N, DTAB), jnp.float32),
            jax.ShapeDtypeStruct((N, 2 * HEADS), jnp.float32),
            jax.ShapeDtypeStruct((1, 2 * HEADS), jnp.float32),
        ],
        scratch_shapes=[
            pltpu.VMEM((1, HEADS), jnp.float32),
            pltpu.VMEM((1, HEADS), jnp.float32),
        ],
    )(x, w, asrc, adst)


def _tc_mid(parts, w, asrc, adst, rexp):
    return pl.pallas_call(
        _tc_mid_body,
        grid=(_GRID,),
        in_specs=[
            pl.BlockSpec((2, _BLK, DROW), lambda i: (0, i, 0)),
            pl.BlockSpec((D_HID, D_HID), lambda i: (0, 0)),
            pl.BlockSpec((D_HID, HEADS), lambda i: (0, 0)),
            pl.BlockSpec((D_HID, HEADS), lambda i: (0, 0)),
            pl.BlockSpec((HEADS, D_HID), lambda i: (0, 0)),
        ],
        out_specs=[
            pl.BlockSpec((_BLK, DTAB), lambda i: (i, 0)),
            pl.BlockSpec((_BLK, 2 * HEADS), lambda i: (i, 0)),
            pl.BlockSpec((1, 2 * HEADS), lambda i: (0, 0)),
        ],
        out_shape=[
            jax.ShapeDtypeStruct((N, DTAB), jnp.float32),
            jax.ShapeDtypeStruct((N, 2 * HEADS), jnp.float32),
            jax.ShapeDtypeStruct((1, 2 * HEADS), jnp.float32),
        ],
        scratch_shapes=[
            pltpu.VMEM((1, HEADS), jnp.float32),
            pltpu.VMEM((1, HEADS), jnp.float32),
        ],
    )(parts, w, asrc, adst, rexp)


def _tc_final(parts, wlin, blin2d, rexp):
    return pl.pallas_call(
        _tc_final_body,
        grid=(_GRID,),
        in_specs=[
            pl.BlockSpec((2, _BLK, DROW), lambda i: (0, i, 0)),
            pl.BlockSpec((D_HID, NUM_CLASSES), lambda i: (0, 0)),
            pl.BlockSpec((1, NUM_CLASSES), lambda i: (0, 0)),
            pl.BlockSpec((HEADS, D_HID), lambda i: (0, 0)),
        ],
        out_specs=pl.BlockSpec((_BLK, NUM_CLASSES), lambda i: (i, 0)),
        out_shape=jax.ShapeDtypeStruct((N, NUM_CLASSES), jnp.float32),
    )(parts, wlin, blin2d, rexp)


def _sc_edges_body(taba, tabb, src, dst, bvec, zeros, out,
                   acc, srcv0, srcv1, srcv2, srcv3, srcv4,
                   dstv0, dstv1, dstv2, dstv3, dstv4,
                   bufa0, bufa1, bufa2, bufa3, bufa4,
                   bufb0, bufb1, bufb2, bufb3, bufb4,
                   bufo0, bufo1, bufo2, bufo3, bufo4, bvv,
                   sema0, sema1, sema2, sema3, sema4,
                   semb0, semb1, semb2, semb3, semb4,
                   semo0, semo1, semo2, semo3, semo4,
                   semi0, semi1, semi2, semi3, semi4):
    srcv = (srcv0, srcv1, srcv2, srcv3, srcv4)
    dstv = (dstv0, dstv1, dstv2, dstv3, dstv4)
    bufa = (bufa0, bufa1, bufa2, bufa3, bufa4)
    bufb = (bufb0, bufb1, bufb2, bufb3, bufb4)
    bufo = (bufo0, bufo1, bufo2, bufo3, bufo4)
    sema = (sema0, sema1, sema2, sema3, sema4)
    semb = (semb0, semb1, semb2, semb3, semb4)
    semo = (semo0, semo1, semo2, semo3, semo4)
    semi = (semi0, semi1, semi2, semi3, semi4)
    E = src.shape[0]
    c = lax.axis_index("c")
    s = lax.axis_index("s")
    rows = (N // NS) & ~7
    tail = N - NS * rows

    per_tile = E // (NC * NS)
    nchunks = per_tile // CH

    pltpu.sync_copy(zeros.at[pl.ds(s * rows, rows)],
                    acc.at[pl.ds(s * rows, rows)])

    @pl.when(s == NS - 1)
    def _():
        pltpu.sync_copy(zeros.at[pl.ds(NS * rows, tail)],
                        acc.at[pl.ds(NS * rows, tail)])

    pltpu.sync_copy(bvec, bvv)
    plsc.subcore_barrier()

    bv = bvv[...]
    lane = lax.iota(jnp.int32, 16)
    hi1 = lane >> 3

    tile = c * NS + s
    base_t = tile * per_tile

    def start_idx(g, b):
        base = pl.multiple_of(base_t + g * CH, 8)
        pltpu.make_async_copy(src.at[pl.ds(base, CH)], srcv[b], semi[b]).start()
        pltpu.make_async_copy(dst.at[pl.ds(base, CH)], dstv[b], semi[b]).start()

    def wait_idx(b):
        pltpu.make_async_copy(src.at[pl.ds(base_t, CH)], srcv[b], semi[b]).wait()
        pltpu.make_async_copy(dst.at[pl.ds(base_t, CH)], dstv[b], semi[b]).wait()

    def start_gathers(b):
        pltpu.async_copy(taba.at[srcv[b]], bufa[b], sema[b])
        pltpu.async_copy(tabb.at[dstv[b]], bufb[b], semb[b])

    def wait_gathers(b):
        pltpu.make_async_copy(taba.at[srcv[b]], bufa[b], sema[b]).wait()
        pltpu.make_async_copy(tabb.at[dstv[b]], bufb[b], semb[b]).wait()

    def start_scatter(b):
        pltpu.make_async_copy(bufo[b], acc.at[dstv[b]], semo[b]).start(add=True)

    def wait_scatter(b):
        pltpu.make_async_copy(bufo[b], acc.at[dstv[b]], semo[b]).wait()

    def compute(b):
        ba, bb, bo = bufa[b], bufb[b], bufo[b]

        @plsc.parallel_loop(0, CH, unroll=4)
        def edge(e):
            sv = ba[e, pl.ds(D_HID, 16)]
            dv = bb[e, pl.ds(0, 16)]
            ev = sv + dv
            ev = jnp.maximum(ev, 0.2 * ev)
            exv = jnp.exp(ev - bv)
            prod0 = None
            for j in range(4):
                hh = ba[e, pl.ds(j * 16, 16)]
                mv = exv.at[2 * j + hi1].get(mode="promise_in_bounds")
                prod = hh * mv
                if j == 0:
                    prod0 = prod
                bo[e, pl.ds(8 + j * 16, 16)] = prod
            perm0 = prod0.at[lane & 7].get(mode="promise_in_bounds")
            bo[e, pl.ds(0, 16)] = jnp.where(lane < 8, exv, perm0)

    start_idx(0, 0)
    start_idx(1, 1)
    start_idx(2, 2)
    start_idx(3, 3)
    wait_idx(0)
    start_gathers(0)
    wait_idx(1)
    start_gathers(1)
    wait_idx(2)
    start_gathers(2)

    def slot_five(g5, carry):
        for b5 in range(5):
            g = g5 * 5 + b5
            b = b5
            b3 = (b5 + 3) % 5
            b4 = (b5 + 4) % 5

            @pl.when(g < nchunks)
            def _():
                @pl.when(g >= 1)
                def _():
                    wait_scatter(b4)

                @pl.when(g + 4 < nchunks)
                def _():
                    start_idx(g + 4, b4)

                @pl.when(g + 3 < nchunks)
                def _():
                    wait_idx(b3)
                    start_gathers(b3)

                wait_gathers(b)
                compute(b)
                start_scatter(b)
        return carry

    lax.fori_loop(0, (nchunks + 4) // 5, slot_five, 0)
    wait_scatter((nchunks - 1) % 5)
    plsc.subcore_barrier()
    pltpu.sync_copy(acc.at[pl.ds(s * rows, rows)],
                    out.at[c, pl.ds(s * rows, rows)])

    @pl.when(s == NS - 1)
    def _():
        pltpu.sync_copy(acc.at[pl.ds(NS * rows, tail)],
                        out.at[c, pl.ds(NS * rows, tail)])


def _sc_edges(taba, tabb, src, dst, bvec, zeros):
    mesh = plsc.VectorSubcoreMesh(core_axis_name="c", subcore_axis_name="s")
    kfn = functools.partial(
        pl.kernel,
        out_type=jax.ShapeDtypeStruct((NC, N, DROW), jnp.float32),
        mesh=mesh,
        compiler_params=pltpu.CompilerParams(use_tc_tiling_on_sc=False),
        scratch_types=(
            [pltpu.VMEM_SHARED((N, DROW), jnp.float32)]
            + [pltpu.VMEM((CH,), jnp.int32)] * 10
            + [pltpu.VMEM((CH, DTAB), jnp.float32)] * 5
            + [pltpu.VMEM((CH, 2 * HEADS), jnp.float32)] * 5
            + [pltpu.VMEM((CH, DROW), jnp.float32)] * 5
            + [pltpu.VMEM((2 * HEADS,), jnp.float32)]
            + [pltpu.SemaphoreType.DMA] * 20
        ),
    )(_sc_edges_body)
    return kfn(taba, tabb, src, dst, bvec, zeros)


def _mix_mat(a):
    eye = jnp.eye(HEADS, dtype=jnp.float32)
    return (eye[:, None, :] * a[:, :, None]).reshape(D_HID, HEADS)


def kernel(x, edge_index, W1, a1_src, a1_dst, Wsem1, bsem1, qsem1,
           W2, a2_src, a2_dst, Wsem2, bsem2, qsem2, Wlin, blin):
    del Wsem1, bsem1, qsem1, Wsem2, bsem2, qsem2
    src = edge_index[0]
    dst = edge_index[1]
    zeros = jnp.zeros((N, DROW), jnp.float32)
    rexp = jnp.kron(jnp.eye(HEADS, dtype=jnp.float32),
                    jnp.ones((1, D_HID // HEADS), jnp.float32))

    taba1, tabb1, bvec1 = _tc_prep(x, W1, _mix_mat(a1_src), _mix_mat(a1_dst))
    parts1 = _sc_edges(taba1, tabb1, src, dst, bvec1.reshape(-1), zeros)
    taba2, tabb2, bvec2 = _tc_mid(parts1, W2, _mix_mat(a2_src),
                                  _mix_mat(a2_dst), rexp)
    parts2 = _sc_edges(taba2, tabb2, src, dst, bvec2.reshape(-1), zeros)
    return _tc_final(parts2, Wlin, blin.reshape(1, -1), rexp)
